# baseline (speedup 1.0000x reference)
import functools
import math

import jax
import jax.numpy as jnp
from jax import lax
from jax.experimental import pallas as pl
from jax.experimental.pallas import tpu as pltpu
from jax.experimental.pallas import tpu_sc as plsc

EMB = 64
SCALE = math.sqrt(EMB)
LANES = 16

NC = 2
NS = 16
NW = NC * NS

G = 128
NBUF = 4


def _body(tok_hbm, row_hbm, table2_hbm, out2_hbm, tok_v, row_v, outbuf,
          *rows_and_sems):
    rows = rows_and_sems[:NBUF]
    sems = rows_and_sems[NBUF:]
    ng = tok_hbm.shape[1]
    per_w2 = ng * (G // 2)

    wid = lax.axis_index("s") * NC + lax.axis_index("c")
    obase = wid * per_w2

    pltpu.sync_copy(tok_hbm.at[wid], tok_v)
    pltpu.sync_copy(row_hbm.at[wid], row_v)

    for b in range(NBUF):
        pltpu.async_copy(table2_hbm.at[row_v.at[b]], rows[b], sems[b])

    iota = lax.iota(jnp.int32, LANES)

    @pl.loop(0, ng, step=NBUF)
    def _chunks(t):
        for b in range(NBUF):
            g = t + b
            pltpu.make_async_copy(
                table2_hbm.at[row_v.at[g]], rows[b], sems[b]
            ).wait()

            for k in range(G // LANES):
                pvec = iota + (k * LANES)
                tvec = tok_v[g, pl.ds(k * LANES, LANES)]
                halfv = (tvec & 1) << 6
                orow = pvec >> 1
                ocol = (pvec & 1) << 6

                @plsc.parallel_loop(0, EMB, unroll=8)
                def _chan(c):
                    vals = plsc.load_gather(rows[b], [pvec, halfv + c])
                    plsc.store_scatter(outbuf, [orow, ocol + c], vals * SCALE)

            pltpu.sync_copy(outbuf, out2_hbm.at[pl.ds(obase + g * (G // 2),
                                                      G // 2)])

            gn = g + NBUF

            @pl.when(gn < ng)
            def _():
                pltpu.async_copy(table2_hbm.at[row_v.at[gn]], rows[b],
                                 sems[b])


def kernel(tokens, table):
    n_tok = tokens.shape[0] * tokens.shape[1]
    per_w = n_tok // NW
    ng = per_w // G

    mesh = plsc.VectorSubcoreMesh(core_axis_name="c", subcore_axis_name="s")
    run = pl.kernel(
        _body,
        out_type=jax.ShapeDtypeStruct((n_tok // 2, 2 * EMB), jnp.float32),
        mesh=mesh,
        compiler_params=pltpu.CompilerParams(needs_layout_passes=False),
        scratch_types=(
            [
                pltpu.VMEM((ng, G), jnp.int32),
                pltpu.VMEM((ng, G), jnp.int32),
                pltpu.VMEM((G // 2, 2 * EMB), jnp.float32),
            ]
            + [pltpu.VMEM((G, 2 * EMB), jnp.float32) for _ in range(NBUF)]
            + [pltpu.SemaphoreType.DMA for _ in range(NBUF)]
        ),
    )
    tok = tokens.reshape(NW, ng, G).astype(jnp.int32)
    row = tok >> 1
    table2 = table.reshape(table.shape[0] // 2, 2 * EMB)
    out2 = run(tok, row, table2)
    return out2.reshape(tokens.shape[0], tokens.shape[1], EMB)


# speedup vs baseline: 2.0954x; 2.0954x over previous
import functools
import math

import jax
import jax.numpy as jnp
from jax import lax
from jax.experimental import pallas as pl
from jax.experimental.pallas import tpu as pltpu
from jax.experimental.pallas import tpu_sc as plsc

EMB = 64
SCALE = math.sqrt(EMB)
LANES = 16

NC = 2
NS = 16
NW = NC * NS

G = 128
NBUF = 4


def _body(tok_hbm, table_hbm, out_hbm, idx_v, *rows_and_sems):
    rows = rows_and_sems[:NBUF]
    sems = rows_and_sems[NBUF:]
    ng = tok_hbm.shape[1]
    per_w = ng * G

    wid = lax.axis_index("s") * NC + lax.axis_index("c")
    base = wid * per_w

    pltpu.sync_copy(tok_hbm.at[wid], idx_v)

    for b in range(NBUF):
        pltpu.async_copy(table_hbm.at[idx_v.at[b]], rows[b], sems[b])

    @pl.loop(0, ng, step=NBUF)
    def _chunks(t):
        for b in range(NBUF):
            g = t + b
            pltpu.make_async_copy(
                table_hbm.at[idx_v.at[g]], rows[b], sems[b]
            ).wait()

            @plsc.parallel_loop(0, G, unroll=4)
            def _scale(r):
                for j in range(EMB // LANES):
                    sl = pl.ds(j * LANES, LANES)
                    rows[b][r, sl] = rows[b][r, sl] * SCALE

            pltpu.sync_copy(
                rows[b],
                out_hbm.at[pl.ds(base + g * G, G), pl.ds(0, EMB)],
            )

            gn = g + NBUF

            @pl.when(gn < ng)
            def _():
                pltpu.async_copy(table_hbm.at[idx_v.at[gn]], rows[b], sems[b])


def kernel(tokens, table):
    n_tok = tokens.shape[0] * tokens.shape[1]
    per_w = n_tok // NW
    ng = per_w // G

    mesh = plsc.VectorSubcoreMesh(core_axis_name="c", subcore_axis_name="s")
    run = pl.kernel(
        _body,
        out_type=jax.ShapeDtypeStruct((n_tok, 2 * EMB), jnp.float32),
        mesh=mesh,
        compiler_params=pltpu.CompilerParams(use_tc_tiling_on_sc=False),
        scratch_types=(
            [pltpu.VMEM((ng, G), jnp.int32)]
            + [pltpu.VMEM((G, EMB), jnp.float32) for _ in range(NBUF)]
            + [pltpu.SemaphoreType.DMA for _ in range(NBUF)]
        ),
    )
    tok = tokens.reshape(NW, ng, G).astype(jnp.int32)
    out = run(tok, table)
    return out[:, :EMB].reshape(tokens.shape[0], tokens.shape[1], EMB)
